# baseline (speedup 1.0000x reference)
import functools
import jax
import jax.numpy as jnp
from jax import lax
from jax.experimental import pallas as pl
from jax.experimental.pallas import tpu as pltpu
from jax.experimental.pallas import tpu_sc as plsc

BATCH = 4096
SEQ = 50
SEQP = 56
DIM = 128
NC, NS = 2, 16
NW = NC * NS
ROWS_PER_W = BATCH // NW
GROUP = 4
GROUPS = ROWS_PER_W // GROUP
NBUF = 4


@functools.partial(
    pl.kernel,
    out_type=jax.ShapeDtypeStruct((BATCH, SEQP, DIM), jnp.float32),
    mesh=plsc.VectorSubcoreMesh(core_axis_name="c", subcore_axis_name="s"),
    scratch_types=(
        [pltpu.VMEM((ROWS_PER_W, SEQ), jnp.int32)]
        + [pltpu.VMEM((GROUP, SEQP, DIM), jnp.float32) for _ in range(NBUF)]
        + [pltpu.SemaphoreType.DMA for _ in range(2 * NBUF)]
    ),
)
def _gather_kernel(x_hbm, table_hbm, out_hbm, idx_v, *scratch):
    bufs = scratch[:NBUF]
    gsem = scratch[NBUF:2 * NBUF]
    ssem = scratch[2 * NBUF:]
    wid = lax.axis_index("s") * NC + lax.axis_index("c")
    base = wid * ROWS_PER_W
    pltpu.sync_copy(x_hbm.at[pl.ds(base, ROWS_PER_W)], idx_v)

    def gather_start(b, g):
        for r in range(GROUP):
            pltpu.async_copy(table_hbm.at[idx_v.at[g * GROUP + r]],
                             bufs[b].at[r].at[pl.ds(0, SEQ)], gsem[b])

    def gather_wait(b, g):
        for r in range(GROUP):
            pltpu.make_async_copy(table_hbm.at[idx_v.at[g * GROUP + r]],
                                  bufs[b].at[r].at[pl.ds(0, SEQ)],
                                  gsem[b]).wait()

    def store_start(b, g):
        pltpu.async_copy(bufs[b],
                         out_hbm.at[pl.ds(base + g * GROUP, GROUP)], ssem[b])

    def store_wait(b, g):
        pltpu.make_async_copy(bufs[b],
                              out_hbm.at[pl.ds(base + g * GROUP, GROUP)],
                              ssem[b]).wait()

    for b in range(NBUF):
        gather_start(b, b)

    def body(t, carry):
        for b in range(NBUF):
            g = t * NBUF + b
            gather_wait(b, g)
            store_start(b, g)
        for b in range(NBUF):
            g = t * NBUF + b
            gn = g + NBUF

            @pl.when(gn < GROUPS)
            def _():
                store_wait(b, g)
                gather_start(b, gn)

        return carry

    lax.fori_loop(0, GROUPS // NBUF, body, 0)
    for b in range(NBUF):
        store_wait(b, GROUPS - NBUF + b)


def kernel(x, table):
    out = _gather_kernel(x.astype(jnp.int32), table)
    return lax.slice(out, (0, 0, 0), (BATCH, SEQ, DIM))
